# baseline (speedup 1.0000x reference)
import functools

import jax
import jax.numpy as jnp
from jax import lax
from jax.experimental import pallas as pl
from jax.experimental.pallas import tpu as pltpu
from jax.experimental.pallas import tpu_sc as plsc

_K = 8192
_D = 32
_BETA = 0.25
_BM = 256
_KC = 2048
_NW = 32


def _argmin_body(nb, nj, m, zb_ref, cb_ref, rowsq_ref, cbn_ref, idx_ref,
                 loss_ref, rmin_ref, rarg_ref, ssq_ref):
    i = pl.program_id(0)
    j = pl.program_id(1)
    zb = zb_ref[...]
    cbj = cb_ref[...]
    xc = lax.dot_general(
        cbj.astype(jnp.bfloat16), zb.astype(jnp.bfloat16),
        (((1,), (1,)), ((), ())),
        preferred_element_type=jnp.float32)
    scores = (rowsq_ref[...] - 2.0 * xc) + cbn_ref[...]
    cmin = jnp.min(scores, axis=0)
    eq = scores == cmin[None, :]
    rowid = lax.broadcasted_iota(jnp.int32, scores.shape, 0)
    carg = jnp.min(jnp.where(eq, rowid, _K), axis=0) + j * _KC

    @pl.when(j == 0)
    def _():
        rmin_ref[...] = jnp.full(rmin_ref.shape, jnp.inf, jnp.float32)
        rarg_ref[...] = jnp.zeros(rarg_ref.shape, jnp.int32)

    take = cmin[None, :] < rmin_ref[...]
    rarg_ref[...] = jnp.where(take, carg[None, :], rarg_ref[...])
    rmin_ref[...] = jnp.where(take, cmin[None, :], rmin_ref[...])

    @pl.when(j == nj - 1)
    def _():
        idx_ref[...] = rarg_ref[...].reshape(idx_ref.shape)

        @pl.when(i == 0)
        def _():
            ssq_ref[0] = 0.0

        ssq_ref[0] += jnp.sum(rmin_ref[...])

        @pl.when(i == nb - 1)
        def _():
            loss_ref[...] = (_BETA * ssq_ref[0] / (m * _D)).reshape(1, 1)


def _tc_argmin(flat, codebook):
    m = flat.shape[0]
    nb = m // _BM
    nj = _K // _KC
    rowsq = jnp.sum(flat * flat, axis=1, keepdims=True).reshape(1, m)
    cbn = jnp.sum(codebook * codebook, axis=1, keepdims=True)
    idx3, loss = pl.pallas_call(
        functools.partial(_argmin_body, nb, nj, m),
        grid=(nb, nj),
        in_specs=[
            pl.BlockSpec((_BM, _D), lambda i, j: (i, 0)),
            pl.BlockSpec((_KC, _D), lambda i, j: (j, 0)),
            pl.BlockSpec((1, _BM), lambda i, j: (0, i)),
            pl.BlockSpec((_KC, 1), lambda i, j: (j, 0)),
        ],
        out_specs=[
            pl.BlockSpec((1, 1, _BM), lambda i, j: (i, 0, 0)),
            pl.BlockSpec((1, 1), lambda i, j: (0, 0)),
        ],
        out_shape=[
            jax.ShapeDtypeStruct((nb, 1, _BM), jnp.int32),
            jax.ShapeDtypeStruct((1, 1), jnp.float32),
        ],
        scratch_shapes=[
            pltpu.VMEM((1, _BM), jnp.float32),
            pltpu.VMEM((1, _BM), jnp.int32),
            pltpu.SMEM((1,), jnp.float32),
        ],
        compiler_params=pltpu.CompilerParams(
            dimension_semantics=("arbitrary", "arbitrary")),
    )(flat, codebook, rowsq, cbn)
    return idx3.reshape(m), loss


def _sc_gather_counts(codebook, idx_flat):
    m = idx_flat.shape[0]
    rows_per = m // _NW
    mesh = plsc.VectorSubcoreMesh(core_axis_name="c", subcore_axis_name="s")

    @functools.partial(
        pl.kernel, mesh=mesh,
        out_type=[
            jax.ShapeDtypeStruct((m, _D), jnp.float32),
            jax.ShapeDtypeStruct((_NW, _K), jnp.float32),
        ],
        scratch_types=[
            pltpu.VMEM((rows_per,), jnp.int32),
            pltpu.VMEM((rows_per, _D), jnp.float32),
            pltpu.VMEM((_K,), jnp.float32),
            pltpu.SemaphoreType.DMA,
        ],
        compiler_params=pltpu.CompilerParams(
            needs_layout_passes=False, use_tc_tiling_on_sc=False),
    )
    def sc_k(cb_hbm, idx_hbm, zq_hbm, cnts_hbm, idx_v, rows_v, cnt_v, sem):
        wid = lax.axis_index("s") * 2 + lax.axis_index("c")
        base = wid * rows_per
        pltpu.sync_copy(idx_hbm.at[pl.ds(base, rows_per)], idx_v)
        pltpu.async_copy(cb_hbm.at[idx_v], rows_v, sem).wait()
        pltpu.sync_copy(rows_v, zq_hbm.at[pl.ds(base, rows_per)])

        def zero_step(t, _):
            cnt_v[pl.ds(t * 16, 16)] = jnp.zeros((16,), jnp.float32)
            return _

        lax.fori_loop(0, _K // 16, zero_step, None)

        ones = jnp.ones((16,), jnp.float32)

        def hist_step(t, _):
            iv = idx_v[pl.ds(t * 16, 16)]
            plsc.addupdate_scatter(cnt_v, [iv], ones)
            return _

        lax.fori_loop(0, rows_per // 16, hist_step, None)
        pltpu.sync_copy(cnt_v, cnts_hbm.at[wid])

    return sc_k(codebook, idx_flat)


def _perp_body(m, cnts_ref, perp_ref):
    c = jnp.sum(cnts_ref[...], axis=0)
    p = c / m
    ent = -jnp.sum(p * jnp.log(p + 1e-10))
    perp_ref[...] = jnp.exp(ent).reshape(1, 1)


def _tc_perplexity(cnts, m):
    return pl.pallas_call(
        functools.partial(_perp_body, m),
        out_shape=jax.ShapeDtypeStruct((1, 1), jnp.float32),
    )(cnts)


def kernel(z, codebook):
    b, t, d = z.shape
    m = b * t
    flat = z.reshape(m, d)
    idx_flat, loss = _tc_argmin(flat, codebook)
    zq, cnts = _sc_gather_counts(codebook, idx_flat)
    perp = _tc_perplexity(cnts, m)
    return (zq.reshape(b, t, d), loss[0, 0], idx_flat.reshape(b, t),
            perp[0, 0])
